# baseline (speedup 1.0000x reference)
import functools

import jax
import jax.numpy as jnp
from jax import lax
from jax.experimental import pallas as pl
from jax.experimental.pallas import tpu as pltpu
from jax.experimental.pallas import tpu_sc as plsc

_N_FIELDS = 26
_VOCAB = 100000
_EMBED_DIM = 16
_BATCH = 16384
_HASH_BUCKETS = 1000

_B = _BATCH * _N_FIELDS
_NW = 32
_BPW = _B // _NW
_C = 1664
_NCH = _BPW // _C


def _hash_body(idx_ref, h_ref, flat_ref):
    x = idx_ref[...]
    f = lax.broadcasted_iota(jnp.int32, x.shape, 1)
    flat_ref[...] = x + f * _VOCAB
    xu = x.astype(jnp.uint32)
    h = (xu * jnp.uint32(2654435761)) % jnp.uint32(_HASH_BUCKETS)
    h_ref[...] = h.astype(jnp.int32)


def _hash_and_flatten(indices):
    return pl.pallas_call(
        _hash_body,
        out_shape=(
            jax.ShapeDtypeStruct((_BATCH, _N_FIELDS), jnp.int32),
            jax.ShapeDtypeStruct((_BATCH, _N_FIELDS), jnp.int32),
        ),
    )(indices)


_MESH = plsc.VectorSubcoreMesh(core_axis_name="c", subcore_axis_name="s")


@functools.partial(
    pl.kernel,
    mesh=_MESH,
    compiler_params=pltpu.CompilerParams(use_tc_tiling_on_sc=False),
    out_type=jax.ShapeDtypeStruct((_B, _EMBED_DIM), jnp.float32),
    scratch_types=[
        pltpu.VMEM((_C,), jnp.int32),
        pltpu.VMEM((_C, _EMBED_DIM), jnp.float32),
        pltpu.SemaphoreType.DMA,
    ],
)
def _sc_gather(idx_hbm, tab_hbm, out_hbm, idx_v, rows_v, gsem):
    wid = lax.axis_index("s") * 2 + lax.axis_index("c")
    base = wid * _BPW

    def chunk(i, carry):
        off = base + i * _C
        pltpu.sync_copy(idx_hbm.at[pl.ds(off, _C)], idx_v)
        pltpu.async_copy(tab_hbm.at[idx_v], rows_v, gsem).wait()
        pltpu.sync_copy(rows_v, out_hbm.at[pl.ds(off, _C)])
        return carry

    lax.fori_loop(0, _NCH, chunk, 0)


def kernel(indices, tables):
    h, idx_flat = _hash_and_flatten(indices)
    tab_flat = tables.reshape(_N_FIELDS * _VOCAB, _EMBED_DIM)
    out_flat = _sc_gather(idx_flat.reshape(_B), tab_flat)
    return out_flat.reshape(_BATCH, _N_FIELDS * _EMBED_DIM), h


# speedup vs baseline: 1.0078x; 1.0078x over previous
import functools

import jax
import jax.numpy as jnp
from jax import lax
from jax.experimental import pallas as pl
from jax.experimental.pallas import tpu as pltpu
from jax.experimental.pallas import tpu_sc as plsc

_N_FIELDS = 26
_VOCAB = 100000
_EMBED_DIM = 16
_BATCH = 16384
_HASH_BUCKETS = 1000

_B = _BATCH * _N_FIELDS
_NW = 32
_BPW = _B // _NW
_C = 1664
_NCH = _BPW // _C


def _hash_body(idx_ref, h_ref, flat_ref):
    x = idx_ref[...]
    f = lax.broadcasted_iota(jnp.int32, x.shape, 1)
    flat_ref[...] = x + f * _VOCAB
    xu = x.astype(jnp.uint32)
    h = (xu * jnp.uint32(2654435761)) % jnp.uint32(_HASH_BUCKETS)
    h_ref[...] = h.astype(jnp.int32)


def _hash_and_flatten(indices):
    return pl.pallas_call(
        _hash_body,
        out_shape=(
            jax.ShapeDtypeStruct((_BATCH, _N_FIELDS), jnp.int32),
            jax.ShapeDtypeStruct((_BATCH, _N_FIELDS), jnp.int32),
        ),
    )(indices)


_MESH = plsc.VectorSubcoreMesh(core_axis_name="c", subcore_axis_name="s")


@functools.partial(
    pl.kernel,
    mesh=_MESH,
    compiler_params=pltpu.CompilerParams(use_tc_tiling_on_sc=False),
    out_type=jax.ShapeDtypeStruct((_B, _EMBED_DIM), jnp.float32),
    scratch_types=[
        pltpu.VMEM((2, _C), jnp.int32),
        pltpu.VMEM((2, _C, _EMBED_DIM), jnp.float32),
        pltpu.SemaphoreType.DMA,
        pltpu.SemaphoreType.DMA,
        pltpu.SemaphoreType.DMA,
        pltpu.SemaphoreType.DMA,
    ],
)
def _sc_gather(idx_hbm, tab_hbm, out_hbm, idx_v, rows_v, g0, g1, s0, s1):
    gsem = (g0, g1)
    ssem = (s0, s1)
    wid = lax.axis_index("s") * 2 + lax.axis_index("c")
    base = wid * _BPW

    gathers = [None] * _NCH
    stores = [None] * _NCH
    for i in range(_NCH):
        b = i % 2
        if i >= 2:
            stores[i - 2].wait()
        off = base + i * _C
        pltpu.sync_copy(idx_hbm.at[pl.ds(off, _C)], idx_v.at[b])
        gathers[i] = pltpu.async_copy(tab_hbm.at[idx_v.at[b]], rows_v.at[b], gsem[b])
        if i >= 1:
            gathers[i - 1].wait()
            poff = base + (i - 1) * _C
            stores[i - 1] = pltpu.async_copy(
                rows_v.at[1 - b], out_hbm.at[pl.ds(poff, _C)], ssem[1 - b])
    gathers[_NCH - 1].wait()
    last = _NCH - 1
    stores[last] = pltpu.async_copy(
        rows_v.at[last % 2], out_hbm.at[pl.ds(base + last * _C, _C)], ssem[last % 2])
    stores[_NCH - 2].wait()
    stores[_NCH - 1].wait()


def kernel(indices, tables):
    h, idx_flat = _hash_and_flatten(indices)
    tab_flat = tables.reshape(_N_FIELDS * _VOCAB, _EMBED_DIM)
    out_flat = _sc_gather(idx_flat.reshape(_B), tab_flat)
    return out_flat.reshape(_BATCH, _N_FIELDS * _EMBED_DIM), h
